# baseline (speedup 1.0000x reference)
import functools

import jax
import jax.numpy as jnp
from jax import lax
from jax.experimental import pallas as pl
from jax.experimental.pallas import tpu as pltpu
from jax.experimental.pallas import tpu_sc as plsc

N = 10000
E = 320000
D = 128

NC = 2
NS = 16
NW = NC * NS

CH = 128
CPW = 80
EPW_PAD = CPW * CH
E_PAD = EPW_PAD * NW
NPAD = 10240
RPT = NPAD // NS


def _edge_kernel_body(*refs):
    (xp, edges, zero128,
     acc_out,
     srcbuf, dstbuf, rows, accsh, sem) = refs

    c = lax.axis_index("c")
    s = lax.axis_index("s")
    w = s * NC + c

    pltpu.sync_copy(zero128.at[pl.ds(s * RPT, RPT)],
                    accsh.at[pl.ds(s * RPT, RPT)])

    pltpu.sync_copy(edges.at[0, w], srcbuf)
    pltpu.sync_copy(edges.at[1, w], dstbuf)
    plsc.subcore_barrier()

    def body(i, carry):
        pltpu.async_copy(xp.at[srcbuf.at[i]], rows, sem).wait()
        pltpu.sync_copy(rows, accsh.at[dstbuf.at[i]], add=True)
        return carry

    lax.fori_loop(0, CPW, body, 0)
    plsc.subcore_barrier()

    pltpu.sync_copy(accsh.at[pl.ds(s * RPT, RPT)],
                    acc_out.at[c, pl.ds(s * RPT, RPT)])


def _make_edge_kernel():
    mesh = plsc.VectorSubcoreMesh(core_axis_name="c", subcore_axis_name="s",
                                  num_cores=NC, num_subcores=NS)
    out_type = [jax.ShapeDtypeStruct((NC, NPAD, D), jnp.float32)]
    scratch = [
        pltpu.VMEM((CPW, CH), jnp.int32),
        pltpu.VMEM((CPW, CH), jnp.int32),
        pltpu.VMEM((CH, D), jnp.float32),
        pltpu.VMEM_SHARED((NPAD, D), jnp.float32),
        pltpu.SemaphoreType.DMA,
    ]
    return pl.kernel(_edge_kernel_body,
                     out_type=out_type, mesh=mesh, scratch_types=scratch)


def _cnt_kernel_body(*refs):
    (dst3, zero128, ones128,
     cnt_out,
     dstbuf, onesbuf, cntsh) = refs

    c = lax.axis_index("c")
    s = lax.axis_index("s")
    w = s * NC + c

    pltpu.sync_copy(zero128.at[pl.ds(s * RPT, RPT)],
                    cntsh.at[pl.ds(s * RPT, RPT)])
    pltpu.sync_copy(ones128, onesbuf)
    pltpu.sync_copy(dst3.at[w], dstbuf)
    plsc.subcore_barrier()

    def body(i, carry):
        pltpu.sync_copy(onesbuf, cntsh.at[dstbuf.at[i]], add=True)
        return carry

    lax.fori_loop(0, CPW, body, 0)
    plsc.subcore_barrier()

    pltpu.sync_copy(cntsh.at[pl.ds(s * RPT, RPT)],
                    cnt_out.at[c, pl.ds(s * RPT, RPT)])


def _make_cnt_kernel():
    mesh = plsc.VectorSubcoreMesh(core_axis_name="c", subcore_axis_name="s",
                                  num_cores=NC, num_subcores=NS)
    out_type = [jax.ShapeDtypeStruct((NC, NPAD, D), jnp.float32)]
    scratch = [
        pltpu.VMEM((CPW, CH), jnp.int32),
        pltpu.VMEM((CH, D), jnp.float32),
        pltpu.VMEM_SHARED((NPAD, D), jnp.float32),
    ]
    return pl.kernel(_cnt_kernel_body,
                     out_type=out_type, mesh=mesh, scratch_types=scratch)


def _proj_body(x_ref, w_ref, b_ref, o_ref):
    o_ref[...] = jax.nn.relu(
        lax.dot_general(x_ref[...], w_ref[...], (((1,), (1,)), ((), ())),
                        preferred_element_type=jnp.float32) + b_ref[...])


def _proj(x, Wp, bp, bn=1000):
    grid = N // bn
    return pl.pallas_call(
        _proj_body,
        grid=(grid,),
        in_specs=[
            pl.BlockSpec((bn, D), lambda i: (i, 0)),
            pl.BlockSpec((D, D), lambda i: (0, 0)),
            pl.BlockSpec((1, D), lambda i: (0, 0)),
        ],
        out_specs=pl.BlockSpec((bn, D), lambda i: (i, 0)),
        out_shape=jax.ShapeDtypeStruct((N, D), jnp.float32),
    )(x, Wp, bp.reshape(1, D))


def _combine_body(fuse_proj, *refs):
    if fuse_proj:
        (acc, cnt, x, wl, bl, wr, g, be, wp, bp, h_out, xp_out) = refs
    else:
        (acc, cnt, x, wl, bl, wr, g, be, h_out) = refs
    a = acc[0] + acc[1]
    cv = cnt[0, :, 0:1] + cnt[1, :, 0:1]
    agg = a / jnp.maximum(cv, 1.0)
    h = (lax.dot_general(agg, wl[...], (((1,), (1,)), ((), ())),
                         preferred_element_type=jnp.float32)
         + bl[...]
         + lax.dot_general(x[...], wr[...], (((1,), (1,)), ((), ())),
                           preferred_element_type=jnp.float32))
    mu = jnp.mean(h, axis=1, keepdims=True)
    var = jnp.mean((h - mu) * (h - mu), axis=1, keepdims=True)
    hn = (h - mu) * lax.rsqrt(var + 1e-5) * g[...] + be[...]
    h_out[...] = hn
    if fuse_proj:
        xp_out[...] = jax.nn.relu(
            lax.dot_general(hn, wp[...], (((1,), (1,)), ((), ())),
                            preferred_element_type=jnp.float32) + bp[...])


def _combine(acc, cnt, x, Wl, bl, Wr, g, be, Wp=None, bp=None, bn=1000):
    fuse = Wp is not None
    grid = N // bn
    row = lambda i: (i, 0)
    full = lambda i: (0, 0)
    in_specs = [
        pl.BlockSpec((NC, bn, D), lambda i: (0, i, 0)),
        pl.BlockSpec((NC, bn, D), lambda i: (0, i, 0)),
        pl.BlockSpec((bn, D), row),
        pl.BlockSpec((D, D), full),
        pl.BlockSpec((1, D), full),
        pl.BlockSpec((D, D), full),
        pl.BlockSpec((1, D), full),
        pl.BlockSpec((1, D), full),
    ]
    args = [acc, cnt, x, Wl, bl.reshape(1, D), Wr,
            g.reshape(1, D), be.reshape(1, D)]
    out_shape = jax.ShapeDtypeStruct((N, D), jnp.float32)
    if fuse:
        in_specs += [pl.BlockSpec((D, D), full), pl.BlockSpec((1, D), full)]
        args += [Wp, bp.reshape(1, D)]
        out_shape = [out_shape, jax.ShapeDtypeStruct((N, D), jnp.float32)]
        out_specs = [pl.BlockSpec((bn, D), row), pl.BlockSpec((bn, D), row)]
    else:
        out_specs = pl.BlockSpec((bn, D), row)
    return pl.pallas_call(
        functools.partial(_combine_body, fuse),
        grid=(grid,),
        in_specs=in_specs,
        out_specs=out_specs,
        out_shape=out_shape,
    )(*args)


def kernel(x, edge_index, Wp0, bp0, Wl0, bl0, Wr0, g0, be0,
           Wp1, bp1, Wl1, bl1, Wr1, g1, be1):
    pad = E_PAD - E
    pad_dst = N + jnp.arange(pad, dtype=jnp.int32) % (NPAD - N)
    srcp = jnp.concatenate([edge_index[0], jnp.zeros((pad,), jnp.int32)])
    dstp = jnp.concatenate([edge_index[1], pad_dst])
    edges = jnp.stack([srcp, dstp]).reshape(2, NW, CPW, CH)
    dst3 = edges[1]

    zero128 = jnp.zeros((NPAD, D), jnp.float32)
    ones128 = jnp.ones((CH, D), jnp.float32)

    edge_kernel = _make_edge_kernel()
    cnt_kernel = _make_cnt_kernel()

    xp0 = _proj(x, Wp0, bp0)
    (cnt,) = cnt_kernel(dst3, zero128, ones128)
    (acc0,) = edge_kernel(xp0, edges, zero128)
    h1, xp1 = _combine(acc0, cnt, x, Wl0, bl0, Wr0, g0, be0, Wp1, bp1)
    (acc1,) = edge_kernel(xp1, edges, zero128)
    out = _combine(acc1, cnt, h1, Wl1, bl1, Wr1, g1, be1)
    return out


# speedup vs baseline: 2.4423x; 2.4423x over previous
import functools

import jax
import jax.numpy as jnp
from jax import lax
from jax.experimental import pallas as pl
from jax.experimental.pallas import tpu as pltpu
from jax.experimental.pallas import tpu_sc as plsc

N = 10000
E = 320000
D = 128

NC = 2
NS = 16
NW = NC * NS

CH = 128
CPW = 80
EPW_PAD = CPW * CH
E_PAD = EPW_PAD * NW
NPAD = 10240
RPT = NPAD // NS


def _edge_kernel_body(*refs):
    (xp, edges, zero128,
     acc_out,
     srcbuf, dstbuf, rows, accsh, sem) = refs

    c = lax.axis_index("c")
    s = lax.axis_index("s")
    w = s * NC + c

    pltpu.sync_copy(zero128.at[pl.ds(s * RPT, RPT)],
                    accsh.at[pl.ds(s * RPT, RPT)])

    pltpu.sync_copy(edges.at[0, w], srcbuf)
    pltpu.sync_copy(edges.at[1, w], dstbuf)
    plsc.subcore_barrier()

    def body(i, carry):
        pltpu.async_copy(xp.at[srcbuf.at[i]], rows, sem).wait()
        pltpu.sync_copy(rows, accsh.at[dstbuf.at[i]], add=True)
        return carry

    lax.fori_loop(0, CPW, body, 0)
    plsc.subcore_barrier()

    pltpu.sync_copy(accsh.at[pl.ds(s * RPT, RPT)],
                    acc_out.at[c, pl.ds(s * RPT, RPT)])


def _make_edge_kernel():
    mesh = plsc.VectorSubcoreMesh(core_axis_name="c", subcore_axis_name="s",
                                  num_cores=NC, num_subcores=NS)
    out_type = [jax.ShapeDtypeStruct((NC, NPAD, D), jnp.float32)]
    scratch = [
        pltpu.VMEM((CPW, CH), jnp.int32),
        pltpu.VMEM((CPW, CH), jnp.int32),
        pltpu.VMEM((CH, D), jnp.float32),
        pltpu.VMEM_SHARED((NPAD, D), jnp.float32),
        pltpu.SemaphoreType.DMA,
    ]
    return pl.kernel(_edge_kernel_body,
                     out_type=out_type, mesh=mesh, scratch_types=scratch)


def _cnt_kernel_body(*refs):
    (dst3, zero128, ones128,
     cnt_out,
     dstbuf, onesbuf, cntsh) = refs

    c = lax.axis_index("c")
    s = lax.axis_index("s")
    w = s * NC + c

    pltpu.sync_copy(zero128.at[pl.ds(s * RPT, RPT)],
                    cntsh.at[pl.ds(s * RPT, RPT)])
    pltpu.sync_copy(ones128, onesbuf)
    pltpu.sync_copy(dst3.at[w], dstbuf)
    plsc.subcore_barrier()

    def body(i, carry):
        pltpu.sync_copy(onesbuf, cntsh.at[dstbuf.at[i]], add=True)
        return carry

    lax.fori_loop(0, CPW, body, 0)
    plsc.subcore_barrier()

    pltpu.sync_copy(cntsh.at[pl.ds(s * RPT, RPT)],
                    cnt_out.at[c, pl.ds(s * RPT, RPT)])


def _make_cnt_kernel():
    mesh = plsc.VectorSubcoreMesh(core_axis_name="c", subcore_axis_name="s",
                                  num_cores=NC, num_subcores=NS)
    out_type = [jax.ShapeDtypeStruct((NC, NPAD, D), jnp.float32)]
    scratch = [
        pltpu.VMEM((CPW, CH), jnp.int32),
        pltpu.VMEM((CH, D), jnp.float32),
        pltpu.VMEM_SHARED((NPAD, D), jnp.float32),
    ]
    return pl.kernel(_cnt_kernel_body,
                     out_type=out_type, mesh=mesh, scratch_types=scratch)


def _proj_body(x_ref, w_ref, b_ref, o_ref):
    o_ref[...] = jax.nn.relu(
        lax.dot_general(x_ref[...], w_ref[...], (((1,), (1,)), ((), ())),
                        preferred_element_type=jnp.float32) + b_ref[...])


def _proj(x, Wp, bp, bn=1000):
    grid = N // bn
    return pl.pallas_call(
        _proj_body,
        grid=(grid,),
        in_specs=[
            pl.BlockSpec((bn, D), lambda i: (i, 0)),
            pl.BlockSpec((D, D), lambda i: (0, 0)),
            pl.BlockSpec((1, D), lambda i: (0, 0)),
        ],
        out_specs=pl.BlockSpec((bn, D), lambda i: (i, 0)),
        out_shape=jax.ShapeDtypeStruct((N, D), jnp.float32),
    )(x, Wp, bp.reshape(1, D))


def _combine_body(fuse_proj, *refs):
    if fuse_proj:
        (acc, cnt, x, wl, bl, wr, g, be, wp, bp, h_out, xp_out) = refs
    else:
        (acc, cnt, x, wl, bl, wr, g, be, h_out) = refs
    a = acc[0] + acc[1]
    cv = cnt[0, :, 0:1] + cnt[1, :, 0:1]
    agg = a / jnp.maximum(cv, 1.0)
    h = (lax.dot_general(agg, wl[...], (((1,), (1,)), ((), ())),
                         preferred_element_type=jnp.float32)
         + bl[...]
         + lax.dot_general(x[...], wr[...], (((1,), (1,)), ((), ())),
                           preferred_element_type=jnp.float32))
    mu = jnp.mean(h, axis=1, keepdims=True)
    var = jnp.mean((h - mu) * (h - mu), axis=1, keepdims=True)
    hn = (h - mu) * lax.rsqrt(var + 1e-5) * g[...] + be[...]
    h_out[...] = hn
    if fuse_proj:
        xp_out[...] = jax.nn.relu(
            lax.dot_general(hn, wp[...], (((1,), (1,)), ((), ())),
                            preferred_element_type=jnp.float32) + bp[...])


def _combine(acc, cnt, x, Wl, bl, Wr, g, be, Wp=None, bp=None, bn=1000):
    fuse = Wp is not None
    grid = N // bn
    row = lambda i: (i, 0)
    full = lambda i: (0, 0)
    in_specs = [
        pl.BlockSpec((NC, bn, D), lambda i: (0, i, 0)),
        pl.BlockSpec((NC, bn, D), lambda i: (0, i, 0)),
        pl.BlockSpec((bn, D), row),
        pl.BlockSpec((D, D), full),
        pl.BlockSpec((1, D), full),
        pl.BlockSpec((D, D), full),
        pl.BlockSpec((1, D), full),
        pl.BlockSpec((1, D), full),
    ]
    args = [acc, cnt, x, Wl, bl.reshape(1, D), Wr,
            g.reshape(1, D), be.reshape(1, D)]
    out_shape = jax.ShapeDtypeStruct((N, D), jnp.float32)
    if fuse:
        in_specs += [pl.BlockSpec((D, D), full), pl.BlockSpec((1, D), full)]
        args += [Wp, bp.reshape(1, D)]
        out_shape = [out_shape, jax.ShapeDtypeStruct((N, D), jnp.float32)]
        out_specs = [pl.BlockSpec((bn, D), row), pl.BlockSpec((bn, D), row)]
    else:
        out_specs = pl.BlockSpec((bn, D), row)
    return pl.pallas_call(
        functools.partial(_combine_body, fuse),
        grid=(grid,),
        in_specs=in_specs,
        out_specs=out_specs,
        out_shape=out_shape,
    )(*args)


def kernel(x, edge_index, Wp0, bp0, Wl0, bl0, Wr0, g0, be0,
           Wp1, bp1, Wl1, bl1, Wr1, g1, be1):
    pad = E_PAD - E
    pad_dst = N + jnp.arange(pad, dtype=jnp.int32) % (NPAD - N)
    pad_src = jnp.arange(pad, dtype=jnp.int32) % N
    srcp = jnp.concatenate([edge_index[0], pad_src])
    dstp = jnp.concatenate([edge_index[1], pad_dst])
    edges = jnp.stack([srcp, dstp]).reshape(2, NW, CPW, CH)
    dst3 = edges[1]

    zero128 = jnp.zeros((NPAD, D), jnp.float32)
    ones128 = jnp.ones((CH, D), jnp.float32)

    edge_kernel = _make_edge_kernel()
    cnt_kernel = _make_cnt_kernel()

    xp0 = _proj(x, Wp0, bp0)
    (cnt,) = cnt_kernel(dst3, zero128, ones128)
    (acc0,) = edge_kernel(xp0, edges, zero128)
    h1, xp1 = _combine(acc0, cnt, x, Wl0, bl0, Wr0, g0, be0, Wp1, bp1)
    (acc1,) = edge_kernel(xp1, edges, zero128)
    out = _combine(acc1, cnt, h1, Wl1, bl1, Wr1, g1, be1)
    return out


# speedup vs baseline: 2.4744x; 1.0131x over previous
import functools

import jax
import jax.numpy as jnp
from jax import lax
from jax.experimental import pallas as pl
from jax.experimental.pallas import tpu as pltpu
from jax.experimental.pallas import tpu_sc as plsc

N = 10000
E = 320000
D = 128

NC = 2
NS = 16
NW = NC * NS

CH = 128
CPW = 79
EPW_PAD = CPW * CH
E_PAD = EPW_PAD * NW
NPAD = 10240
RPT = NPAD // NS


def _edge_kernel_body(*refs):
    (xp, edges, zero128,
     acc_out,
     srcbuf, dstbuf, rows, accsh, sem) = refs

    c = lax.axis_index("c")
    s = lax.axis_index("s")
    w = s * NC + c

    pltpu.sync_copy(zero128.at[pl.ds(s * RPT, RPT)],
                    accsh.at[pl.ds(s * RPT, RPT)])

    pltpu.sync_copy(edges.at[0, w], srcbuf)
    pltpu.sync_copy(edges.at[1, w], dstbuf)
    plsc.subcore_barrier()

    def body(i, carry):
        pltpu.async_copy(xp.at[srcbuf.at[i]], rows, sem).wait()
        pltpu.sync_copy(rows, accsh.at[dstbuf.at[i]], add=True)
        return carry

    lax.fori_loop(0, CPW, body, 0)
    plsc.subcore_barrier()

    pltpu.sync_copy(accsh.at[pl.ds(s * RPT, RPT)],
                    acc_out.at[c, pl.ds(s * RPT, RPT)])


def _make_edge_kernel():
    mesh = plsc.VectorSubcoreMesh(core_axis_name="c", subcore_axis_name="s",
                                  num_cores=NC, num_subcores=NS)
    out_type = [jax.ShapeDtypeStruct((NC, NPAD, D), jnp.float32)]
    scratch = [
        pltpu.VMEM((CPW, CH), jnp.int32),
        pltpu.VMEM((CPW, CH), jnp.int32),
        pltpu.VMEM((CH, D), jnp.float32),
        pltpu.VMEM_SHARED((NPAD, D), jnp.float32),
        pltpu.SemaphoreType.DMA,
    ]
    return pl.kernel(_edge_kernel_body,
                     out_type=out_type, mesh=mesh, scratch_types=scratch)


def _cnt_kernel_body(*refs):
    (dst3, zero128, ones128,
     cnt_out,
     dstbuf, onesbuf, cntsh) = refs

    c = lax.axis_index("c")
    s = lax.axis_index("s")
    w = s * NC + c

    pltpu.sync_copy(zero128.at[pl.ds(s * RPT, RPT)],
                    cntsh.at[pl.ds(s * RPT, RPT)])
    pltpu.sync_copy(ones128, onesbuf)
    pltpu.sync_copy(dst3.at[w], dstbuf)
    plsc.subcore_barrier()

    def body(i, carry):
        pltpu.sync_copy(onesbuf, cntsh.at[dstbuf.at[i]], add=True)
        return carry

    lax.fori_loop(0, CPW, body, 0)
    plsc.subcore_barrier()

    pltpu.sync_copy(cntsh.at[pl.ds(s * RPT, RPT)],
                    cnt_out.at[c, pl.ds(s * RPT, RPT)])


def _make_cnt_kernel():
    mesh = plsc.VectorSubcoreMesh(core_axis_name="c", subcore_axis_name="s",
                                  num_cores=NC, num_subcores=NS)
    out_type = [jax.ShapeDtypeStruct((NC, NPAD, D), jnp.float32)]
    scratch = [
        pltpu.VMEM((CPW, CH), jnp.int32),
        pltpu.VMEM((CH, D), jnp.float32),
        pltpu.VMEM_SHARED((NPAD, D), jnp.float32),
    ]
    return pl.kernel(_cnt_kernel_body,
                     out_type=out_type, mesh=mesh, scratch_types=scratch)


def _proj_body(x_ref, w_ref, b_ref, o_ref):
    o_ref[...] = jax.nn.relu(
        lax.dot_general(x_ref[...], w_ref[...], (((1,), (1,)), ((), ())),
                        preferred_element_type=jnp.float32) + b_ref[...])


def _proj(x, Wp, bp, bn=1000):
    grid = N // bn
    return pl.pallas_call(
        _proj_body,
        grid=(grid,),
        in_specs=[
            pl.BlockSpec((bn, D), lambda i: (i, 0)),
            pl.BlockSpec((D, D), lambda i: (0, 0)),
            pl.BlockSpec((1, D), lambda i: (0, 0)),
        ],
        out_specs=pl.BlockSpec((bn, D), lambda i: (i, 0)),
        out_shape=jax.ShapeDtypeStruct((N, D), jnp.float32),
    )(x, Wp, bp.reshape(1, D))


def _combine_body(fuse_proj, *refs):
    if fuse_proj:
        (acc, cnt, x, wl, bl, wr, g, be, wp, bp, h_out, xp_out) = refs
    else:
        (acc, cnt, x, wl, bl, wr, g, be, h_out) = refs
    a = acc[0] + acc[1]
    cv = cnt[0, :, 0:1] + cnt[1, :, 0:1]
    agg = a / jnp.maximum(cv, 1.0)
    h = (lax.dot_general(agg, wl[...], (((1,), (1,)), ((), ())),
                         preferred_element_type=jnp.float32)
         + bl[...]
         + lax.dot_general(x[...], wr[...], (((1,), (1,)), ((), ())),
                           preferred_element_type=jnp.float32))
    mu = jnp.mean(h, axis=1, keepdims=True)
    var = jnp.mean((h - mu) * (h - mu), axis=1, keepdims=True)
    hn = (h - mu) * lax.rsqrt(var + 1e-5) * g[...] + be[...]
    h_out[...] = hn
    if fuse_proj:
        xp_out[...] = jax.nn.relu(
            lax.dot_general(hn, wp[...], (((1,), (1,)), ((), ())),
                            preferred_element_type=jnp.float32) + bp[...])


def _combine(acc, cnt, x, Wl, bl, Wr, g, be, Wp=None, bp=None, bn=1000):
    fuse = Wp is not None
    grid = N // bn
    row = lambda i: (i, 0)
    full = lambda i: (0, 0)
    in_specs = [
        pl.BlockSpec((NC, bn, D), lambda i: (0, i, 0)),
        pl.BlockSpec((NC, bn, D), lambda i: (0, i, 0)),
        pl.BlockSpec((bn, D), row),
        pl.BlockSpec((D, D), full),
        pl.BlockSpec((1, D), full),
        pl.BlockSpec((D, D), full),
        pl.BlockSpec((1, D), full),
        pl.BlockSpec((1, D), full),
    ]
    args = [acc, cnt, x, Wl, bl.reshape(1, D), Wr,
            g.reshape(1, D), be.reshape(1, D)]
    out_shape = jax.ShapeDtypeStruct((N, D), jnp.float32)
    if fuse:
        in_specs += [pl.BlockSpec((D, D), full), pl.BlockSpec((1, D), full)]
        args += [Wp, bp.reshape(1, D)]
        out_shape = [out_shape, jax.ShapeDtypeStruct((N, D), jnp.float32)]
        out_specs = [pl.BlockSpec((bn, D), row), pl.BlockSpec((bn, D), row)]
    else:
        out_specs = pl.BlockSpec((bn, D), row)
    return pl.pallas_call(
        functools.partial(_combine_body, fuse),
        grid=(grid,),
        in_specs=in_specs,
        out_specs=out_specs,
        out_shape=out_shape,
    )(*args)


def kernel(x, edge_index, Wp0, bp0, Wl0, bl0, Wr0, g0, be0,
           Wp1, bp1, Wl1, bl1, Wr1, g1, be1):
    pad = E_PAD - E
    pad_dst = N + jnp.arange(pad, dtype=jnp.int32) % (NPAD - N)
    pad_src = jnp.arange(pad, dtype=jnp.int32) % N
    srcp = jnp.concatenate([edge_index[0], pad_src])
    dstp = jnp.concatenate([edge_index[1], pad_dst])
    edges = jnp.stack([srcp, dstp]).reshape(2, NW, CPW, CH)
    dst3 = edges[1]

    zero128 = jnp.zeros((NPAD, D), jnp.float32)
    ones128 = jnp.ones((CH, D), jnp.float32)

    edge_kernel = _make_edge_kernel()
    cnt_kernel = _make_cnt_kernel()

    xp0 = _proj(x, Wp0, bp0)
    (cnt,) = cnt_kernel(dst3, zero128, ones128)
    (acc0,) = edge_kernel(xp0, edges, zero128)
    h1, xp1 = _combine(acc0, cnt, x, Wl0, bl0, Wr0, g0, be0, Wp1, bp1)
    (acc1,) = edge_kernel(xp1, edges, zero128)
    out = _combine(acc1, cnt, h1, Wl1, bl1, Wr1, g1, be1)
    return out
